# Initial kernel scaffold; baseline (speedup 1.0000x reference)
import functools

import jax
import jax.numpy as jnp
from jax import lax
from jax.experimental import pallas as pl
from jax.experimental.pallas import tpu as pltpu
from jax.experimental.pallas import tpu_sc as plsc

N_NODES = 10000
N_EDGES = 320000
D_IN = 128
HID = 128
N_CLS = 2
N_GRP = 64

NC = 2
NS = 16
NW = NC * NS
EPW = N_EDGES // NW
K = 128
NCH = EPW // K
TAIL = EPW - NCH * K
RPT = N_NODES // NS

_MESH = plsc.VectorSubcoreMesh(core_axis_name="c", subcore_axis_name="s")


@functools.partial(
    pl.kernel,
    out_type=jax.ShapeDtypeStruct((NC, N_NODES, 16), jnp.float32),
    mesh=_MESH,
    scratch_types=[
        pltpu.VMEM((K,), jnp.int32),
        pltpu.VMEM((K, 16), jnp.float32),
        pltpu.VMEM((TAIL,), jnp.int32),
        pltpu.VMEM((TAIL, 16), jnp.float32),
        pltpu.VMEM_SHARED((N_NODES, 16), jnp.float32),
    ],
)
def _sc_deg(dst_hbm, zeros_hbm, ones_hbm, out_hbm, dstb, ones, dstt, onest, acc):
    c = lax.axis_index("c")
    s = lax.axis_index("s")
    wid = s * NC + c
    base = wid * EPW
    r0 = s * RPT
    pltpu.sync_copy(ones_hbm, ones)
    pltpu.sync_copy(ones_hbm.at[pl.ds(0, TAIL)], onest)
    pltpu.sync_copy(zeros_hbm.at[pl.ds(r0, RPT)], acc.at[pl.ds(r0, RPT)])
    plsc.subcore_barrier()

    def chunk(j, carry):
        st = pl.multiple_of(base + j * K, 8)
        pltpu.sync_copy(dst_hbm.at[pl.ds(st, K)], dstb)
        pltpu.sync_copy(ones, acc.at[dstb], add=True)
        return carry

    lax.fori_loop(0, NCH, chunk, 0)
    stt = pl.multiple_of(base + NCH * K, 8)
    pltpu.sync_copy(dst_hbm.at[pl.ds(stt, TAIL)], dstt)
    pltpu.sync_copy(onest, acc.at[dstt], add=True)
    plsc.subcore_barrier()
    pltpu.sync_copy(acc.at[pl.ds(r0, RPT)], out_hbm.at[c, pl.ds(r0, RPT)])


@functools.partial(
    pl.kernel,
    out_type=jax.ShapeDtypeStruct((NC, N_NODES, HID), jnp.float32),
    mesh=_MESH,
    scratch_types=[
        pltpu.VMEM((K,), jnp.int32),
        pltpu.VMEM((K,), jnp.int32),
        pltpu.VMEM((K, HID), jnp.float32),
        pltpu.VMEM((TAIL,), jnp.int32),
        pltpu.VMEM((TAIL,), jnp.int32),
        pltpu.VMEM((TAIL, HID), jnp.float32),
        pltpu.VMEM_SHARED((N_NODES, HID), jnp.float32),
        pltpu.SemaphoreType.DMA,
    ],
)
def _sc_scatter(src_hbm, dst_hbm, y_hbm, zeros_hbm, out_hbm,
                srcb, dstb, rows, srct, dstt, rowst, acc, sem):
    c = lax.axis_index("c")
    s = lax.axis_index("s")
    wid = s * NC + c
    base = wid * EPW
    r0 = s * RPT
    pltpu.sync_copy(zeros_hbm.at[pl.ds(r0, RPT)], acc.at[pl.ds(r0, RPT)])
    plsc.subcore_barrier()

    def chunk(j, carry):
        st = pl.multiple_of(base + j * K, 8)
        pltpu.sync_copy(src_hbm.at[pl.ds(st, K)], srcb)
        pltpu.sync_copy(dst_hbm.at[pl.ds(st, K)], dstb)
        pltpu.async_copy(y_hbm.at[srcb], rows, sem).wait()
        pltpu.sync_copy(rows, acc.at[dstb], add=True)
        return carry

    lax.fori_loop(0, NCH, chunk, 0)
    stt = pl.multiple_of(base + NCH * K, 8)
    pltpu.sync_copy(src_hbm.at[pl.ds(stt, TAIL)], srct)
    pltpu.sync_copy(dst_hbm.at[pl.ds(stt, TAIL)], dstt)
    pltpu.async_copy(y_hbm.at[srct], rowst, sem).wait()
    pltpu.sync_copy(rowst, acc.at[dstt], add=True)
    plsc.subcore_barrier()
    pltpu.sync_copy(acc.at[pl.ds(r0, RPT)], out_hbm.at[c, pl.ds(r0, RPT)])


BN = 1000
GRID = N_NODES // BN


def _tc1_body(x_ref, w_ref, dp_ref, y_ref, dinv_ref):
    deg = dp_ref[0, :, :1] + dp_ref[1, :, :1] + 1.0
    di = lax.rsqrt(deg)
    dinv_ref[...] = di
    xw = jnp.dot(x_ref[...], w_ref[...], preferred_element_type=jnp.float32)
    y_ref[...] = xw * di


def _tc1(x, W1, deg_parts):
    return pl.pallas_call(
        _tc1_body,
        grid=(GRID,),
        in_specs=[
            pl.BlockSpec((BN, D_IN), lambda i: (i, 0)),
            pl.BlockSpec((D_IN, HID), lambda i: (0, 0)),
            pl.BlockSpec((NC, BN, 16), lambda i: (0, i, 0)),
        ],
        out_specs=[
            pl.BlockSpec((BN, HID), lambda i: (i, 0)),
            pl.BlockSpec((BN, 1), lambda i: (i, 0)),
        ],
        out_shape=[
            jax.ShapeDtypeStruct((N_NODES, HID), jnp.float32),
            jax.ShapeDtypeStruct((N_NODES, 1), jnp.float32),
        ],
    )(x, W1, deg_parts)


def _tc2_body(s_ref, y1_ref, dinv_ref, b1_ref, w2_ref, y2_ref):
    di = dinv_ref[...]
    h = s_ref[0] + s_ref[1] + y1_ref[...]
    h = jnp.maximum(di * h + b1_ref[...][None, :], 0.0)
    y2_ref[...] = di * jnp.dot(h, w2_ref[...], preferred_element_type=jnp.float32)


def _tc2(s1, y1, dinv, b1, W2):
    return pl.pallas_call(
        _tc2_body,
        grid=(GRID,),
        in_specs=[
            pl.BlockSpec((NC, BN, HID), lambda i: (0, i, 0)),
            pl.BlockSpec((BN, HID), lambda i: (i, 0)),
            pl.BlockSpec((BN, 1), lambda i: (i, 0)),
            pl.BlockSpec((HID,), lambda i: (0,)),
            pl.BlockSpec((HID, HID), lambda i: (0, 0)),
        ],
        out_specs=pl.BlockSpec((BN, HID), lambda i: (i, 0)),
        out_shape=jax.ShapeDtypeStruct((N_NODES, HID), jnp.float32),
    )(s1, y1, dinv, b1, W2)


def _tc3_body(s_ref, y2_ref, dinv_ref, b2_ref, batch_ref, wl_ref, bl_ref,
              out_ref, acc_ref):
    i = pl.program_id(0)

    @pl.when(i == 0)
    def _zero():
        acc_ref[...] = jnp.zeros_like(acc_ref)

    di = dinv_ref[...]
    h2 = di * (s_ref[0] + s_ref[1] + y2_ref[...]) + b2_ref[...][None, :]
    bt = batch_ref[...][:, 0]
    m = (bt[None, :] == lax.broadcasted_iota(jnp.int32, (N_GRP, BN), 0))
    acc_ref[...] += jnp.dot(m.astype(jnp.float32), h2,
                            preferred_element_type=jnp.float32)

    @pl.when(i == GRID - 1)
    def _final():
        out_ref[...] = (jnp.dot(acc_ref[...], wl_ref[...],
                                preferred_element_type=jnp.float32)
                        + bl_ref[...][None, :])


def _tc3(s2, y2, dinv, b2, batch2d, Wl, bl):
    return pl.pallas_call(
        _tc3_body,
        grid=(GRID,),
        in_specs=[
            pl.BlockSpec((NC, BN, HID), lambda i: (0, i, 0)),
            pl.BlockSpec((BN, HID), lambda i: (i, 0)),
            pl.BlockSpec((BN, 1), lambda i: (i, 0)),
            pl.BlockSpec((HID,), lambda i: (0,)),
            pl.BlockSpec((BN, 1), lambda i: (i, 0)),
            pl.BlockSpec((HID, N_CLS), lambda i: (0, 0)),
            pl.BlockSpec((N_CLS,), lambda i: (0,)),
        ],
        out_specs=pl.BlockSpec((N_GRP, N_CLS), lambda i: (0, 0)),
        out_shape=jax.ShapeDtypeStruct((N_GRP, N_CLS), jnp.float32),
        scratch_shapes=[pltpu.VMEM((N_GRP, HID), jnp.float32)],
    )(s2, y2, dinv, b2, batch2d, Wl, bl)


def kernel(x, edge_index, batch, W1, b1, W2, b2, Wl, bl):
    src = edge_index[0]
    dst = edge_index[1]
    zeros_nh = jnp.zeros((N_NODES, HID), jnp.float32)
    zeros_16 = jnp.zeros((N_NODES, 16), jnp.float32)
    ones_k16 = jnp.ones((K, 16), jnp.float32)

    deg_parts = _sc_deg(dst, zeros_16, ones_k16)
    y1, dinv = _tc1(x, W1, deg_parts)
    s1 = _sc_scatter(src, dst, y1, zeros_nh)
    y2 = _tc2(s1, y1, dinv, b1, W2)
    s2 = _sc_scatter(src, dst, y2, zeros_nh)
    return _tc3(s2, y2, dinv, b2, batch.reshape(N_NODES, 1), Wl, bl)


# speedup vs baseline: 14.9439x; 14.9439x over previous
import functools

import jax
import jax.numpy as jnp
from jax import lax
from jax.experimental import pallas as pl
from jax.experimental.pallas import tpu as pltpu
from jax.experimental.pallas import tpu_sc as plsc

N_NODES = 10000
N_EDGES = 320000
D_IN = 128
HID = 128
N_CLS = 2
N_GRP = 64

NC = 2
NS = 16
NW = NC * NS
EPW = N_EDGES // NW
K = 128
NCH = EPW // K
TAIL = EPW - NCH * K
RPT = 624
REM_OFF = RPT * NS
REM = N_NODES - REM_OFF

_MESH = plsc.VectorSubcoreMesh(core_axis_name="c", subcore_axis_name="s")


@functools.partial(
    pl.kernel,
    out_type=jax.ShapeDtypeStruct((NC * N_NODES,), jnp.float32),
    mesh=_MESH,
    scratch_types=[
        pltpu.VMEM((K,), jnp.int32),
        pltpu.VMEM((K,), jnp.float32),
        pltpu.VMEM((TAIL,), jnp.int32),
        pltpu.VMEM((TAIL,), jnp.float32),
        pltpu.VMEM((RPT,), jnp.float32),
        pltpu.VMEM_SHARED((N_NODES,), jnp.float32),
    ],
)
def _sc_deg(dst_hbm, out_hbm, dstb, ones, dstt, onest, stage, acc):
    c = lax.axis_index("c")
    s = lax.axis_index("s")
    wid = s * NC + c
    base = wid * EPW
    r0 = s * RPT
    for i in range(K // 16):
        ones[pl.ds(i * 16, 16)] = jnp.full((16,), 1.0, jnp.float32)
    onest[...] = jnp.full((TAIL,), 1.0, jnp.float32)

    def zs(i, carry):
        stage[pl.ds(i * 16, 16)] = jnp.zeros((16,), jnp.float32)
        return carry

    lax.fori_loop(0, RPT // 16, zs, 0)
    pltpu.sync_copy(stage, acc.at[pl.ds(r0, RPT)])

    @pl.when(s == NS - 1)
    def _zero_rem():
        pltpu.sync_copy(stage.at[pl.ds(0, REM)], acc.at[pl.ds(REM_OFF, REM)])

    plsc.subcore_barrier()

    def chunk(j, carry):
        st = pl.multiple_of(base + j * K, 8)
        pltpu.sync_copy(dst_hbm.at[pl.ds(st, K)], dstb)
        pltpu.sync_copy(ones, acc.at[dstb], add=True)
        return carry

    lax.fori_loop(0, NCH, chunk, 0)
    stt = pl.multiple_of(base + NCH * K, 8)
    pltpu.sync_copy(dst_hbm.at[pl.ds(stt, TAIL)], dstt)
    pltpu.sync_copy(onest, acc.at[dstt], add=True)
    plsc.subcore_barrier()
    pltpu.sync_copy(acc.at[pl.ds(r0, RPT)], stage)
    ob = pl.multiple_of(c * N_NODES + r0, 8)
    pltpu.sync_copy(stage, out_hbm.at[pl.ds(ob, RPT)])

    @pl.when(s == NS - 1)
    def _out_rem():
        pltpu.sync_copy(acc.at[pl.ds(REM_OFF, REM)], onest)
        ob2 = pl.multiple_of(c * N_NODES + REM_OFF, 8)
        pltpu.sync_copy(onest, out_hbm.at[pl.ds(ob2, REM)])


@functools.partial(
    pl.kernel,
    out_type=jax.ShapeDtypeStruct((NC, N_NODES, HID), jnp.float32),
    mesh=_MESH,
    scratch_types=[
        pltpu.VMEM((K,), jnp.int32),
        pltpu.VMEM((K,), jnp.int32),
        pltpu.VMEM((K, HID), jnp.float32),
        pltpu.VMEM((TAIL,), jnp.int32),
        pltpu.VMEM((TAIL,), jnp.int32),
        pltpu.VMEM((TAIL, HID), jnp.float32),
        pltpu.VMEM_SHARED((N_NODES, HID), jnp.float32),
        pltpu.SemaphoreType.DMA,
    ],
)
def _sc_scatter(src_hbm, dst_hbm, y_hbm, zeros_hbm, out_hbm,
                srcb, dstb, rows, srct, dstt, rowst, acc, sem):
    c = lax.axis_index("c")
    s = lax.axis_index("s")
    wid = s * NC + c
    base = wid * EPW
    r0 = s * RPT
    pltpu.sync_copy(zeros_hbm.at[pl.ds(r0, RPT)], acc.at[pl.ds(r0, RPT)])

    @pl.when(s == NS - 1)
    def _zero_rem():
        pltpu.sync_copy(zeros_hbm.at[pl.ds(REM_OFF, REM)],
                        acc.at[pl.ds(REM_OFF, REM)])

    plsc.subcore_barrier()

    def chunk(j, carry):
        st = pl.multiple_of(base + j * K, 8)
        pltpu.sync_copy(src_hbm.at[pl.ds(st, K)], srcb)
        pltpu.sync_copy(dst_hbm.at[pl.ds(st, K)], dstb)
        pltpu.async_copy(y_hbm.at[srcb], rows, sem).wait()
        pltpu.sync_copy(rows, acc.at[dstb], add=True)
        return carry

    lax.fori_loop(0, NCH, chunk, 0)
    stt = pl.multiple_of(base + NCH * K, 8)
    pltpu.sync_copy(src_hbm.at[pl.ds(stt, TAIL)], srct)
    pltpu.sync_copy(dst_hbm.at[pl.ds(stt, TAIL)], dstt)
    pltpu.async_copy(y_hbm.at[srct], rowst, sem).wait()
    pltpu.sync_copy(rowst, acc.at[dstt], add=True)
    plsc.subcore_barrier()
    pltpu.sync_copy(acc.at[pl.ds(r0, RPT)], out_hbm.at[c, pl.ds(r0, RPT)])

    @pl.when(s == NS - 1)
    def _out_rem():
        pltpu.sync_copy(acc.at[pl.ds(REM_OFF, REM)],
                        out_hbm.at[c, pl.ds(REM_OFF, REM)])


BN = 1000
GRID = N_NODES // BN


def _tc1_body(x_ref, w_ref, dp_ref, y_ref, dinv_ref):
    deg = dp_ref[0] + dp_ref[1] + 1.0
    di = lax.rsqrt(deg)
    dinv_ref[...] = di
    xw = jnp.dot(x_ref[...], w_ref[...], preferred_element_type=jnp.float32)
    y_ref[...] = xw * di


def _tc1(x, W1, deg_parts):
    return pl.pallas_call(
        _tc1_body,
        grid=(GRID,),
        in_specs=[
            pl.BlockSpec((BN, D_IN), lambda i: (i, 0)),
            pl.BlockSpec((D_IN, HID), lambda i: (0, 0)),
            pl.BlockSpec((NC, BN, 1), lambda i: (0, i, 0)),
        ],
        out_specs=[
            pl.BlockSpec((BN, HID), lambda i: (i, 0)),
            pl.BlockSpec((BN, 1), lambda i: (i, 0)),
        ],
        out_shape=[
            jax.ShapeDtypeStruct((N_NODES, HID), jnp.float32),
            jax.ShapeDtypeStruct((N_NODES, 1), jnp.float32),
        ],
    )(x, W1, deg_parts)


def _tc2_body(s_ref, y1_ref, dinv_ref, b1_ref, w2_ref, y2_ref):
    di = dinv_ref[...]
    h = s_ref[0] + s_ref[1] + y1_ref[...]
    h = jnp.maximum(di * h + b1_ref[...][None, :], 0.0)
    y2_ref[...] = di * jnp.dot(h, w2_ref[...], preferred_element_type=jnp.float32)


def _tc2(s1, y1, dinv, b1, W2):
    return pl.pallas_call(
        _tc2_body,
        grid=(GRID,),
        in_specs=[
            pl.BlockSpec((NC, BN, HID), lambda i: (0, i, 0)),
            pl.BlockSpec((BN, HID), lambda i: (i, 0)),
            pl.BlockSpec((BN, 1), lambda i: (i, 0)),
            pl.BlockSpec((HID,), lambda i: (0,)),
            pl.BlockSpec((HID, HID), lambda i: (0, 0)),
        ],
        out_specs=pl.BlockSpec((BN, HID), lambda i: (i, 0)),
        out_shape=jax.ShapeDtypeStruct((N_NODES, HID), jnp.float32),
    )(s1, y1, dinv, b1, W2)


def _tc3_body(s_ref, y2_ref, dinv_ref, b2_ref, batch_ref, wl_ref, bl_ref,
              out_ref, acc_ref):
    i = pl.program_id(0)

    @pl.when(i == 0)
    def _zero():
        acc_ref[...] = jnp.zeros_like(acc_ref)

    di = dinv_ref[...]
    h2 = di * (s_ref[0] + s_ref[1] + y2_ref[...]) + b2_ref[...][None, :]
    bt = batch_ref[...][:, 0]
    m = (bt[None, :] == lax.broadcasted_iota(jnp.int32, (N_GRP, BN), 0))
    acc_ref[...] += jnp.dot(m.astype(jnp.float32), h2,
                            preferred_element_type=jnp.float32)

    @pl.when(i == GRID - 1)
    def _final():
        out_ref[...] = (jnp.dot(acc_ref[...], wl_ref[...],
                                preferred_element_type=jnp.float32)
                        + bl_ref[...][None, :])


def _tc3(s2, y2, dinv, b2, batch2d, Wl, bl):
    return pl.pallas_call(
        _tc3_body,
        grid=(GRID,),
        in_specs=[
            pl.BlockSpec((NC, BN, HID), lambda i: (0, i, 0)),
            pl.BlockSpec((BN, HID), lambda i: (i, 0)),
            pl.BlockSpec((BN, 1), lambda i: (i, 0)),
            pl.BlockSpec((HID,), lambda i: (0,)),
            pl.BlockSpec((BN, 1), lambda i: (i, 0)),
            pl.BlockSpec((HID, N_CLS), lambda i: (0, 0)),
            pl.BlockSpec((N_CLS,), lambda i: (0,)),
        ],
        out_specs=pl.BlockSpec((N_GRP, N_CLS), lambda i: (0, 0)),
        out_shape=jax.ShapeDtypeStruct((N_GRP, N_CLS), jnp.float32),
        scratch_shapes=[pltpu.VMEM((N_GRP, HID), jnp.float32)],
    )(s2, y2, dinv, b2, batch2d, Wl, bl)


def kernel(x, edge_index, batch, W1, b1, W2, b2, Wl, bl):
    src = edge_index[0]
    dst = edge_index[1]
    zeros_nh = jnp.zeros((N_NODES, HID), jnp.float32)

    deg_parts = _sc_deg(dst).reshape(NC, N_NODES, 1)
    y1, dinv = _tc1(x, W1, deg_parts)
    s1 = _sc_scatter(src, dst, y1, zeros_nh)
    y2 = _tc2(s1, y1, dinv, b1, W2)
    s2 = _sc_scatter(src, dst, y2, zeros_nh)
    return _tc3(s2, y2, dinv, b2, batch.reshape(N_NODES, 1), Wl, bl)


# speedup vs baseline: 20.7529x; 1.3887x over previous
import functools

import jax
import jax.numpy as jnp
from jax import lax
from jax.experimental import pallas as pl
from jax.experimental.pallas import tpu as pltpu
from jax.experimental.pallas import tpu_sc as plsc

N_NODES = 10000
N_EDGES = 320000
D_IN = 128
HID = 128
N_CLS = 2
N_GRP = 64

NC = 2
NS = 16
NW = NC * NS
EPW = N_EDGES // NW
K = 128
NCH = EPW // K
TAIL = EPW - NCH * K
RPT = 624
REM_OFF = RPT * NS
REM = N_NODES - REM_OFF

_MESH = plsc.VectorSubcoreMesh(core_axis_name="c", subcore_axis_name="s")


@functools.partial(
    pl.kernel,
    out_type=jax.ShapeDtypeStruct((NC * N_NODES,), jnp.float32),
    mesh=_MESH,
    scratch_types=[
        pltpu.VMEM((K,), jnp.int32),
        pltpu.VMEM((K,), jnp.float32),
        pltpu.VMEM((TAIL,), jnp.int32),
        pltpu.VMEM((TAIL,), jnp.float32),
        pltpu.VMEM((RPT,), jnp.float32),
        pltpu.VMEM_SHARED((N_NODES,), jnp.float32),
    ],
)
def _sc_deg(dst_hbm, out_hbm, dstb, ones, dstt, onest, stage, acc):
    c = lax.axis_index("c")
    s = lax.axis_index("s")
    wid = s * NC + c
    base = wid * EPW
    r0 = s * RPT
    for i in range(K // 16):
        ones[pl.ds(i * 16, 16)] = jnp.full((16,), 1.0, jnp.float32)
    onest[...] = jnp.full((TAIL,), 1.0, jnp.float32)

    def zs(i, carry):
        stage[pl.ds(i * 16, 16)] = jnp.zeros((16,), jnp.float32)
        return carry

    lax.fori_loop(0, RPT // 16, zs, 0)
    pltpu.sync_copy(stage, acc.at[pl.ds(r0, RPT)])

    @pl.when(s == NS - 1)
    def _zero_rem():
        pltpu.sync_copy(stage.at[pl.ds(0, REM)], acc.at[pl.ds(REM_OFF, REM)])

    plsc.subcore_barrier()

    def chunk(j, carry):
        st = pl.multiple_of(base + j * K, 8)
        pltpu.sync_copy(dst_hbm.at[pl.ds(st, K)], dstb)
        pltpu.sync_copy(ones, acc.at[dstb], add=True)
        return carry

    lax.fori_loop(0, NCH, chunk, 0)
    stt = pl.multiple_of(base + NCH * K, 8)
    pltpu.sync_copy(dst_hbm.at[pl.ds(stt, TAIL)], dstt)
    pltpu.sync_copy(onest, acc.at[dstt], add=True)
    plsc.subcore_barrier()
    pltpu.sync_copy(acc.at[pl.ds(r0, RPT)], stage)
    ob = pl.multiple_of(c * N_NODES + r0, 8)
    pltpu.sync_copy(stage, out_hbm.at[pl.ds(ob, RPT)])

    @pl.when(s == NS - 1)
    def _out_rem():
        pltpu.sync_copy(acc.at[pl.ds(REM_OFF, REM)], onest)
        ob2 = pl.multiple_of(c * N_NODES + REM_OFF, 8)
        pltpu.sync_copy(onest, out_hbm.at[pl.ds(ob2, REM)])


@functools.partial(
    pl.kernel,
    out_type=jax.ShapeDtypeStruct((NC, N_NODES, HID), jnp.float32),
    mesh=_MESH,
    scratch_types=[
        pltpu.VMEM((K,), jnp.int32),
        pltpu.VMEM((K,), jnp.int32),
        pltpu.VMEM((K, HID), jnp.float32),
        pltpu.VMEM((K,), jnp.int32),
        pltpu.VMEM((K,), jnp.int32),
        pltpu.VMEM((K, HID), jnp.float32),
        pltpu.VMEM((TAIL,), jnp.int32),
        pltpu.VMEM((TAIL,), jnp.int32),
        pltpu.VMEM((TAIL, HID), jnp.float32),
        pltpu.VMEM_SHARED((N_NODES, HID), jnp.float32),
        pltpu.SemaphoreType.DMA,
        pltpu.SemaphoreType.DMA,
    ],
)
def _sc_scatter(src_hbm, dst_hbm, y_hbm, zeros_hbm, out_hbm,
                srcb0, dstb0, rows0, srcb1, dstb1, rows1,
                srct, dstt, rowst, acc, sem0, sem1):
    c = lax.axis_index("c")
    s = lax.axis_index("s")
    wid = s * NC + c
    base = wid * EPW
    r0 = s * RPT
    pltpu.sync_copy(zeros_hbm.at[pl.ds(r0, RPT)], acc.at[pl.ds(r0, RPT)])

    @pl.when(s == NS - 1)
    def _zero_rem():
        pltpu.sync_copy(zeros_hbm.at[pl.ds(REM_OFF, REM)],
                        acc.at[pl.ds(REM_OFF, REM)])

    plsc.subcore_barrier()

    bufs = ((srcb0, dstb0, rows0, sem0), (srcb1, dstb1, rows1, sem1))

    def load_and_gather(j, b):
        srcb, dstb, rows, sem = bufs[b]
        st = pl.multiple_of(base + j * K, 8)
        pltpu.sync_copy(src_hbm.at[pl.ds(st, K)], srcb)
        pltpu.sync_copy(dst_hbm.at[pl.ds(st, K)], dstb)
        pltpu.async_copy(y_hbm.at[srcb], rows, sem)

    def drain_and_scatter(b):
        srcb, dstb, rows, sem = bufs[b]
        pltpu.make_async_copy(y_hbm.at[srcb], rows, sem).wait()
        pltpu.sync_copy(rows, acc.at[dstb], add=True)

    load_and_gather(0, 0)

    def pair(jj, carry):
        j0 = jj * 2
        load_and_gather(j0 + 1, 1)
        drain_and_scatter(0)

        @pl.when(j0 + 2 < NCH)
        def _next():
            load_and_gather(j0 + 2, 0)

        drain_and_scatter(1)
        return carry

    lax.fori_loop(0, NCH // 2, pair, 0)
    stt = pl.multiple_of(base + NCH * K, 8)
    pltpu.sync_copy(src_hbm.at[pl.ds(stt, TAIL)], srct)
    pltpu.sync_copy(dst_hbm.at[pl.ds(stt, TAIL)], dstt)
    pltpu.async_copy(y_hbm.at[srct], rowst, sem0).wait()
    pltpu.sync_copy(rowst, acc.at[dstt], add=True)
    plsc.subcore_barrier()
    pltpu.sync_copy(acc.at[pl.ds(r0, RPT)], out_hbm.at[c, pl.ds(r0, RPT)])

    @pl.when(s == NS - 1)
    def _out_rem():
        pltpu.sync_copy(acc.at[pl.ds(REM_OFF, REM)],
                        out_hbm.at[c, pl.ds(REM_OFF, REM)])


BN = 1000
GRID = N_NODES // BN


def _tc1_body(x_ref, w_ref, dp_ref, y_ref, dinv_ref):
    deg = dp_ref[0] + dp_ref[1] + 1.0
    di = lax.rsqrt(deg)
    dinv_ref[...] = di
    xw = jnp.dot(x_ref[...], w_ref[...], preferred_element_type=jnp.float32)
    y_ref[...] = xw * di


def _tc1(x, W1, deg_parts):
    return pl.pallas_call(
        _tc1_body,
        grid=(GRID,),
        in_specs=[
            pl.BlockSpec((BN, D_IN), lambda i: (i, 0)),
            pl.BlockSpec((D_IN, HID), lambda i: (0, 0)),
            pl.BlockSpec((NC, BN, 1), lambda i: (0, i, 0)),
        ],
        out_specs=[
            pl.BlockSpec((BN, HID), lambda i: (i, 0)),
            pl.BlockSpec((BN, 1), lambda i: (i, 0)),
        ],
        out_shape=[
            jax.ShapeDtypeStruct((N_NODES, HID), jnp.float32),
            jax.ShapeDtypeStruct((N_NODES, 1), jnp.float32),
        ],
    )(x, W1, deg_parts)


def _tc2_body(s_ref, y1_ref, dinv_ref, b1_ref, w2_ref, y2_ref):
    di = dinv_ref[...]
    h = s_ref[0] + s_ref[1] + y1_ref[...]
    h = jnp.maximum(di * h + b1_ref[...][None, :], 0.0)
    y2_ref[...] = di * jnp.dot(h, w2_ref[...], preferred_element_type=jnp.float32)


def _tc2(s1, y1, dinv, b1, W2):
    return pl.pallas_call(
        _tc2_body,
        grid=(GRID,),
        in_specs=[
            pl.BlockSpec((NC, BN, HID), lambda i: (0, i, 0)),
            pl.BlockSpec((BN, HID), lambda i: (i, 0)),
            pl.BlockSpec((BN, 1), lambda i: (i, 0)),
            pl.BlockSpec((HID,), lambda i: (0,)),
            pl.BlockSpec((HID, HID), lambda i: (0, 0)),
        ],
        out_specs=pl.BlockSpec((BN, HID), lambda i: (i, 0)),
        out_shape=jax.ShapeDtypeStruct((N_NODES, HID), jnp.float32),
    )(s1, y1, dinv, b1, W2)


def _tc3_body(s_ref, y2_ref, dinv_ref, b2_ref, batch_ref, wl_ref, bl_ref,
              out_ref, acc_ref):
    i = pl.program_id(0)

    @pl.when(i == 0)
    def _zero():
        acc_ref[...] = jnp.zeros_like(acc_ref)

    di = dinv_ref[...]
    h2 = di * (s_ref[0] + s_ref[1] + y2_ref[...]) + b2_ref[...][None, :]
    bt = batch_ref[...][:, 0]
    m = (bt[None, :] == lax.broadcasted_iota(jnp.int32, (N_GRP, BN), 0))
    acc_ref[...] += jnp.dot(m.astype(jnp.float32), h2,
                            preferred_element_type=jnp.float32,
                            precision=lax.Precision.HIGHEST)

    @pl.when(i == GRID - 1)
    def _final():
        out_ref[...] = (jnp.dot(acc_ref[...], wl_ref[...],
                                preferred_element_type=jnp.float32)
                        + bl_ref[...][None, :])


def _tc3(s2, y2, dinv, b2, batch2d, Wl, bl):
    return pl.pallas_call(
        _tc3_body,
        grid=(GRID,),
        in_specs=[
            pl.BlockSpec((NC, BN, HID), lambda i: (0, i, 0)),
            pl.BlockSpec((BN, HID), lambda i: (i, 0)),
            pl.BlockSpec((BN, 1), lambda i: (i, 0)),
            pl.BlockSpec((HID,), lambda i: (0,)),
            pl.BlockSpec((BN, 1), lambda i: (i, 0)),
            pl.BlockSpec((HID, N_CLS), lambda i: (0, 0)),
            pl.BlockSpec((N_CLS,), lambda i: (0,)),
        ],
        out_specs=pl.BlockSpec((N_GRP, N_CLS), lambda i: (0, 0)),
        out_shape=jax.ShapeDtypeStruct((N_GRP, N_CLS), jnp.float32),
        scratch_shapes=[pltpu.VMEM((N_GRP, HID), jnp.float32)],
    )(s2, y2, dinv, b2, batch2d, Wl, bl)


def kernel(x, edge_index, batch, W1, b1, W2, b2, Wl, bl):
    src = edge_index[0]
    dst = edge_index[1]
    zeros_nh = jnp.zeros((N_NODES, HID), jnp.float32)

    deg_parts = _sc_deg(dst).reshape(NC, N_NODES, 1)
    y1, dinv = _tc1(x, W1, deg_parts)
    s1 = _sc_scatter(src, dst, y1, zeros_nh)
    y2 = _tc2(s1, y1, dinv, b1, W2)
    s2 = _sc_scatter(src, dst, y2, zeros_nh)
    return _tc3(s2, y2, dinv, b2, batch.reshape(N_NODES, 1), Wl, bl)


# speedup vs baseline: 24.0020x; 1.1566x over previous
import functools

import jax
import jax.numpy as jnp
from jax import lax
from jax.experimental import pallas as pl
from jax.experimental.pallas import tpu as pltpu
from jax.experimental.pallas import tpu_sc as plsc

N_NODES = 10000
N_EDGES = 320000
D_IN = 128
HID = 128
N_CLS = 2
N_GRP = 64

NC = 2
NS = 16
NW = NC * NS
EPW = N_EDGES // NW
K = 128
NCH = EPW // K
TAIL = EPW - NCH * K
RPT = 624
REM_OFF = RPT * NS
REM = N_NODES - REM_OFF

_MESH = plsc.VectorSubcoreMesh(core_axis_name="c", subcore_axis_name="s")


@functools.partial(
    pl.kernel,
    out_type=jax.ShapeDtypeStruct((NC * N_NODES,), jnp.float32),
    mesh=_MESH,
    scratch_types=[
        pltpu.VMEM((K,), jnp.int32),
        pltpu.VMEM((K,), jnp.int32),
        pltpu.VMEM((K,), jnp.int32),
        pltpu.VMEM((K,), jnp.float32),
        pltpu.VMEM((TAIL,), jnp.int32),
        pltpu.VMEM((TAIL,), jnp.float32),
        pltpu.VMEM((RPT,), jnp.float32),
        pltpu.VMEM_SHARED((N_NODES,), jnp.float32),
        pltpu.SemaphoreType.DMA,
        pltpu.SemaphoreType.DMA,
        pltpu.SemaphoreType.DMA,
    ],
)
def _sc_deg(dst_hbm, out_hbm, dstb0, dstb1, dstb2, ones, dstt, onest, stage,
            acc, sem0, sem1, sem2):
    c = lax.axis_index("c")
    s = lax.axis_index("s")
    wid = s * NC + c
    base = pl.multiple_of(wid * EPW, 8)
    r0 = s * RPT
    for i in range(K // 16):
        ones[pl.ds(i * 16, 16)] = jnp.full((16,), 1.0, jnp.float32)
    onest[...] = jnp.full((TAIL,), 1.0, jnp.float32)

    def zs(i, carry):
        stage[pl.ds(i * 16, 16)] = jnp.zeros((16,), jnp.float32)
        return carry

    lax.fori_loop(0, RPT // 16, zs, 0)
    pltpu.sync_copy(stage, acc.at[pl.ds(r0, RPT)])

    @pl.when(s == NS - 1)
    def _zero_rem():
        pltpu.sync_copy(stage.at[pl.ds(0, REM)], acc.at[pl.ds(REM_OFF, REM)])

    plsc.subcore_barrier()

    bufs = ((dstb0, sem0), (dstb1, sem1), (dstb2, sem2))
    NB = len(bufs)

    def prefetch(j, b):
        dstb, sem = bufs[b]
        st = pl.multiple_of(base + j * K, 8)
        pltpu.async_copy(dst_hbm.at[pl.ds(st, K)], dstb, sem)

    for b in range(NB):
        prefetch(b, b)

    def round3(jj, carry):
        j0 = jj * NB
        for b in range(NB):
            dstb, sem = bufs[b]
            pltpu.make_async_copy(dst_hbm.at[pl.ds(0, K)], dstb, sem).wait()
            pltpu.sync_copy(ones, acc.at[dstb], add=True)

            @pl.when(j0 + b + NB < NCH)
            def _next():
                prefetch(j0 + b + NB, b)

        return carry

    lax.fori_loop(0, NCH // NB, round3, 0)
    stt = pl.multiple_of(base + NCH * K, 8)
    pltpu.sync_copy(dst_hbm.at[pl.ds(stt, TAIL)], dstt)
    pltpu.sync_copy(onest, acc.at[dstt], add=True)
    plsc.subcore_barrier()
    pltpu.sync_copy(acc.at[pl.ds(r0, RPT)], stage)
    ob = pl.multiple_of(c * N_NODES + r0, 8)
    pltpu.sync_copy(stage, out_hbm.at[pl.ds(ob, RPT)])

    @pl.when(s == NS - 1)
    def _out_rem():
        pltpu.sync_copy(acc.at[pl.ds(REM_OFF, REM)], onest)
        ob2 = pl.multiple_of(c * N_NODES + REM_OFF, 8)
        pltpu.sync_copy(onest, out_hbm.at[pl.ds(ob2, REM)])


@functools.partial(
    pl.kernel,
    out_type=jax.ShapeDtypeStruct((NC, N_NODES, HID), jnp.float32),
    mesh=_MESH,
    scratch_types=[
        pltpu.VMEM((K,), jnp.int32),
        pltpu.VMEM((K,), jnp.int32),
        pltpu.VMEM((K,), jnp.int32),
        pltpu.VMEM((K,), jnp.int32),
        pltpu.VMEM((K,), jnp.int32),
        pltpu.VMEM((K,), jnp.int32),
        pltpu.VMEM((K, HID), jnp.float32),
        pltpu.VMEM((K, HID), jnp.float32),
        pltpu.VMEM((K, HID), jnp.float32),
        pltpu.VMEM((TAIL,), jnp.int32),
        pltpu.VMEM((TAIL,), jnp.int32),
        pltpu.VMEM_SHARED((N_NODES, HID), jnp.float32),
        pltpu.SemaphoreType.DMA,
        pltpu.SemaphoreType.DMA,
        pltpu.SemaphoreType.DMA,
        pltpu.SemaphoreType.DMA,
        pltpu.SemaphoreType.DMA,
        pltpu.SemaphoreType.DMA,
    ],
)
def _sc_scatter(src_hbm, dst_hbm, y_hbm, zeros_hbm, out_hbm,
                srcb0, srcb1, srcb2, dstb0, dstb1, dstb2,
                rows0, rows1, rows2, srct, dstt, acc,
                gi0, gi1, gi2, gg0, gg1, gg2):
    c = lax.axis_index("c")
    s = lax.axis_index("s")
    wid = s * NC + c
    base = pl.multiple_of(wid * EPW, 8)
    r0 = s * RPT
    pltpu.sync_copy(zeros_hbm.at[pl.ds(r0, RPT)], acc.at[pl.ds(r0, RPT)])

    @pl.when(s == NS - 1)
    def _zero_rem():
        pltpu.sync_copy(zeros_hbm.at[pl.ds(REM_OFF, REM)],
                        acc.at[pl.ds(REM_OFF, REM)])

    plsc.subcore_barrier()

    bufs = ((srcb0, dstb0, rows0, gi0, gg0),
            (srcb1, dstb1, rows1, gi1, gg1),
            (srcb2, dstb2, rows2, gi2, gg2))
    NB = len(bufs)

    def idx_start(j, b):
        srcb, dstb, rows, gi, gg = bufs[b]
        st = pl.multiple_of(base + j * K, 8)
        pltpu.async_copy(src_hbm.at[pl.ds(st, K)], srcb, gi)
        pltpu.async_copy(dst_hbm.at[pl.ds(st, K)], dstb, gi)

    def idx_wait_gather_start(b):
        srcb, dstb, rows, gi, gg = bufs[b]
        pltpu.make_async_copy(src_hbm.at[pl.ds(0, K)], srcb, gi).wait()
        pltpu.make_async_copy(dst_hbm.at[pl.ds(0, K)], dstb, gi).wait()
        pltpu.async_copy(y_hbm.at[srcb], rows, gg)

    def gather_wait(b):
        srcb, dstb, rows, gi, gg = bufs[b]
        pltpu.make_async_copy(y_hbm.at[srcb], rows, gg).wait()

    def scatter(b):
        srcb, dstb, rows, gi, gg = bufs[b]
        pltpu.sync_copy(rows, acc.at[dstb], add=True)

    for b in range(NB):
        idx_start(b, b)
    idx_wait_gather_start(0)

    def round3(jj, carry):
        j0 = jj * NB
        for b in range(NB):
            j = j0 + b
            gather_wait(b)

            @pl.when(j + 1 < NCH)
            def _g():
                idx_wait_gather_start((b + 1) % NB)

            scatter(b)

            @pl.when(j + NB < NCH)
            def _i():
                idx_start(j + NB, b)

        return carry

    lax.fori_loop(0, NCH // NB, round3, 0)
    stt = pl.multiple_of(base + NCH * K, 8)
    pltpu.sync_copy(src_hbm.at[pl.ds(stt, TAIL)], srct)
    pltpu.sync_copy(dst_hbm.at[pl.ds(stt, TAIL)], dstt)
    pltpu.async_copy(y_hbm.at[srct], rows0.at[pl.ds(0, TAIL)], gg0).wait()
    pltpu.sync_copy(rows0.at[pl.ds(0, TAIL)], acc.at[dstt], add=True)
    plsc.subcore_barrier()
    pltpu.sync_copy(acc.at[pl.ds(r0, RPT)], out_hbm.at[c, pl.ds(r0, RPT)])

    @pl.when(s == NS - 1)
    def _out_rem():
        pltpu.sync_copy(acc.at[pl.ds(REM_OFF, REM)],
                        out_hbm.at[c, pl.ds(REM_OFF, REM)])


BN = 1000
GRID = N_NODES // BN


def _tc1_body(x_ref, w_ref, dp_ref, y_ref, dinv_ref):
    deg = dp_ref[0] + dp_ref[1] + 1.0
    di = lax.rsqrt(deg)
    dinv_ref[...] = di
    xw = jnp.dot(x_ref[...], w_ref[...], preferred_element_type=jnp.float32)
    y_ref[...] = xw * di


def _tc1(x, W1, deg_parts):
    return pl.pallas_call(
        _tc1_body,
        grid=(GRID,),
        in_specs=[
            pl.BlockSpec((BN, D_IN), lambda i: (i, 0)),
            pl.BlockSpec((D_IN, HID), lambda i: (0, 0)),
            pl.BlockSpec((NC, BN, 1), lambda i: (0, i, 0)),
        ],
        out_specs=[
            pl.BlockSpec((BN, HID), lambda i: (i, 0)),
            pl.BlockSpec((BN, 1), lambda i: (i, 0)),
        ],
        out_shape=[
            jax.ShapeDtypeStruct((N_NODES, HID), jnp.float32),
            jax.ShapeDtypeStruct((N_NODES, 1), jnp.float32),
        ],
    )(x, W1, deg_parts)


def _tc2_body(s_ref, y1_ref, dinv_ref, b1_ref, w2_ref, y2_ref):
    di = dinv_ref[...]
    h = s_ref[0] + s_ref[1] + y1_ref[...]
    h = jnp.maximum(di * h + b1_ref[...][None, :], 0.0)
    y2_ref[...] = di * jnp.dot(h, w2_ref[...], preferred_element_type=jnp.float32)


def _tc2(s1, y1, dinv, b1, W2):
    return pl.pallas_call(
        _tc2_body,
        grid=(GRID,),
        in_specs=[
            pl.BlockSpec((NC, BN, HID), lambda i: (0, i, 0)),
            pl.BlockSpec((BN, HID), lambda i: (i, 0)),
            pl.BlockSpec((BN, 1), lambda i: (i, 0)),
            pl.BlockSpec((HID,), lambda i: (0,)),
            pl.BlockSpec((HID, HID), lambda i: (0, 0)),
        ],
        out_specs=pl.BlockSpec((BN, HID), lambda i: (i, 0)),
        out_shape=jax.ShapeDtypeStruct((N_NODES, HID), jnp.float32),
    )(s1, y1, dinv, b1, W2)


def _tc3_body(s_ref, y2_ref, dinv_ref, b2_ref, batch_ref, wl_ref, bl_ref,
              out_ref, acc_ref):
    i = pl.program_id(0)

    @pl.when(i == 0)
    def _zero():
        acc_ref[...] = jnp.zeros_like(acc_ref)

    di = dinv_ref[...]
    h2 = di * (s_ref[0] + s_ref[1] + y2_ref[...]) + b2_ref[...][None, :]
    bt = batch_ref[...][:, 0]
    m = (bt[None, :] == lax.broadcasted_iota(jnp.int32, (N_GRP, BN), 0))
    acc_ref[...] += jnp.dot(m.astype(jnp.float32), h2,
                            preferred_element_type=jnp.float32,
                            precision=lax.Precision.HIGHEST)

    @pl.when(i == GRID - 1)
    def _final():
        out_ref[...] = (jnp.dot(acc_ref[...], wl_ref[...],
                                preferred_element_type=jnp.float32)
                        + bl_ref[...][None, :])


def _tc3(s2, y2, dinv, b2, batch2d, Wl, bl):
    return pl.pallas_call(
        _tc3_body,
        grid=(GRID,),
        in_specs=[
            pl.BlockSpec((NC, BN, HID), lambda i: (0, i, 0)),
            pl.BlockSpec((BN, HID), lambda i: (i, 0)),
            pl.BlockSpec((BN, 1), lambda i: (i, 0)),
            pl.BlockSpec((HID,), lambda i: (0,)),
            pl.BlockSpec((BN, 1), lambda i: (i, 0)),
            pl.BlockSpec((HID, N_CLS), lambda i: (0, 0)),
            pl.BlockSpec((N_CLS,), lambda i: (0,)),
        ],
        out_specs=pl.BlockSpec((N_GRP, N_CLS), lambda i: (0, 0)),
        out_shape=jax.ShapeDtypeStruct((N_GRP, N_CLS), jnp.float32),
        scratch_shapes=[pltpu.VMEM((N_GRP, HID), jnp.float32)],
    )(s2, y2, dinv, b2, batch2d, Wl, bl)


def kernel(x, edge_index, batch, W1, b1, W2, b2, Wl, bl):
    src = edge_index[0]
    dst = edge_index[1]
    zeros_nh = jnp.zeros((N_NODES, HID), jnp.float32)

    deg_parts = _sc_deg(dst).reshape(NC, N_NODES, 1)
    y1, dinv = _tc1(x, W1, deg_parts)
    s1 = _sc_scatter(src, dst, y1, zeros_nh)
    y2 = _tc2(s1, y1, dinv, b1, W2)
    s2 = _sc_scatter(src, dst, y2, zeros_nh)
    return _tc3(s2, y2, dinv, b2, batch.reshape(N_NODES, 1), Wl, bl)


# speedup vs baseline: 27.8042x; 1.1584x over previous
import functools

import jax
import jax.numpy as jnp
from jax import lax
from jax.experimental import pallas as pl
from jax.experimental.pallas import tpu as pltpu
from jax.experimental.pallas import tpu_sc as plsc

N_NODES = 10000
N_EDGES = 320000
D_IN = 128
HID = 128
N_CLS = 2
N_GRP = 64

NC = 2
NS = 16
NW = NC * NS
EPW = N_EDGES // NW
K = 128
NCH = EPW // K
TAIL = EPW - NCH * K
RPT = 624
REM_OFF = RPT * NS
REM = N_NODES - REM_OFF

_MESH = plsc.VectorSubcoreMesh(core_axis_name="c", subcore_axis_name="s")


@functools.partial(
    pl.kernel,
    out_type=jax.ShapeDtypeStruct((NC * N_NODES,), jnp.float32),
    mesh=_MESH,
    scratch_types=[
        pltpu.VMEM((K,), jnp.int32),
        pltpu.VMEM((K,), jnp.int32),
        pltpu.VMEM((K,), jnp.int32),
        pltpu.VMEM((K,), jnp.float32),
        pltpu.VMEM((TAIL,), jnp.int32),
        pltpu.VMEM((TAIL,), jnp.float32),
        pltpu.VMEM((RPT,), jnp.float32),
        pltpu.VMEM_SHARED((N_NODES,), jnp.float32),
        pltpu.SemaphoreType.DMA,
        pltpu.SemaphoreType.DMA,
        pltpu.SemaphoreType.DMA,
    ],
)
def _sc_deg(dst_hbm, out_hbm, dstb0, dstb1, dstb2, ones, dstt, onest, stage,
            acc, sem0, sem1, sem2):
    c = lax.axis_index("c")
    s = lax.axis_index("s")
    wid = s * NC + c
    base = pl.multiple_of(wid * EPW, 8)
    r0 = s * RPT
    for i in range(K // 16):
        ones[pl.ds(i * 16, 16)] = jnp.full((16,), 1.0, jnp.float32)
    onest[...] = jnp.full((TAIL,), 1.0, jnp.float32)

    def zs(i, carry):
        stage[pl.ds(i * 16, 16)] = jnp.zeros((16,), jnp.float32)
        return carry

    lax.fori_loop(0, RPT // 16, zs, 0)
    pltpu.sync_copy(stage, acc.at[pl.ds(r0, RPT)])

    @pl.when(s == NS - 1)
    def _zero_rem():
        pltpu.sync_copy(stage.at[pl.ds(0, REM)], acc.at[pl.ds(REM_OFF, REM)])

    plsc.subcore_barrier()

    bufs = ((dstb0, sem0), (dstb1, sem1), (dstb2, sem2))
    NB = len(bufs)

    def prefetch(j, b):
        dstb, sem = bufs[b]
        st = pl.multiple_of(base + j * K, 8)
        pltpu.async_copy(dst_hbm.at[pl.ds(st, K)], dstb, sem)

    for b in range(NB):
        prefetch(b, b)

    def round3(jj, carry):
        j0 = jj * NB
        for b in range(NB):
            dstb, sem = bufs[b]
            pltpu.make_async_copy(dst_hbm.at[pl.ds(0, K)], dstb, sem).wait()
            pltpu.sync_copy(ones, acc.at[dstb], add=True)

            @pl.when(j0 + b + NB < NCH)
            def _next():
                prefetch(j0 + b + NB, b)

        return carry

    lax.fori_loop(0, NCH // NB, round3, 0)
    stt = pl.multiple_of(base + NCH * K, 8)
    pltpu.sync_copy(dst_hbm.at[pl.ds(stt, TAIL)], dstt)
    pltpu.sync_copy(onest, acc.at[dstt], add=True)
    plsc.subcore_barrier()
    pltpu.sync_copy(acc.at[pl.ds(r0, RPT)], stage)
    ob = pl.multiple_of(c * N_NODES + r0, 8)
    pltpu.sync_copy(stage, out_hbm.at[pl.ds(ob, RPT)])

    @pl.when(s == NS - 1)
    def _out_rem():
        pltpu.sync_copy(acc.at[pl.ds(REM_OFF, REM)], onest)
        ob2 = pl.multiple_of(c * N_NODES + REM_OFF, 8)
        pltpu.sync_copy(onest, out_hbm.at[pl.ds(ob2, REM)])


@functools.partial(
    pl.kernel,
    out_type=jax.ShapeDtypeStruct((NC, N_NODES, HID), jnp.float32),
    mesh=_MESH,
    scratch_types=[
        pltpu.VMEM((K,), jnp.int32),
        pltpu.VMEM((K,), jnp.int32),
        pltpu.VMEM((K,), jnp.int32),
        pltpu.VMEM((K,), jnp.int32),
        pltpu.VMEM((K,), jnp.int32),
        pltpu.VMEM((K,), jnp.int32),
        pltpu.VMEM((K, HID), jnp.float32),
        pltpu.VMEM((K, HID), jnp.float32),
        pltpu.VMEM((K, HID), jnp.float32),
        pltpu.VMEM((TAIL,), jnp.int32),
        pltpu.VMEM((TAIL,), jnp.int32),
        pltpu.VMEM_SHARED((N_NODES, HID), jnp.float32),
        pltpu.SemaphoreType.DMA,
        pltpu.SemaphoreType.DMA,
        pltpu.SemaphoreType.DMA,
        pltpu.SemaphoreType.DMA,
        pltpu.SemaphoreType.DMA,
        pltpu.SemaphoreType.DMA,
    ],
)
def _sc_scatter(src_hbm, dst_hbm, y_hbm, zeros_hbm, out_hbm,
                srcb0, srcb1, srcb2, dstb0, dstb1, dstb2,
                rows0, rows1, rows2, srct, dstt, acc,
                gi0, gi1, gi2, gg0, gg1, gg2):
    c = lax.axis_index("c")
    s = lax.axis_index("s")
    wid = s * NC + c
    base = pl.multiple_of(wid * EPW, 8)
    r0 = s * RPT
    pltpu.sync_copy(zeros_hbm.at[pl.ds(r0, RPT)], acc.at[pl.ds(r0, RPT)])

    @pl.when(s == NS - 1)
    def _zero_rem():
        pltpu.sync_copy(zeros_hbm.at[pl.ds(REM_OFF, REM)],
                        acc.at[pl.ds(REM_OFF, REM)])

    plsc.subcore_barrier()

    bufs = ((srcb0, dstb0, rows0, gi0, gg0),
            (srcb1, dstb1, rows1, gi1, gg1),
            (srcb2, dstb2, rows2, gi2, gg2))
    NB = len(bufs)

    def idx_start(j, b):
        srcb, dstb, rows, gi, gg = bufs[b]
        st = pl.multiple_of(base + j * K, 8)
        pltpu.async_copy(src_hbm.at[pl.ds(st, K)], srcb, gi)
        pltpu.async_copy(dst_hbm.at[pl.ds(st, K)], dstb, gi)

    def idx_wait_gather_start(b):
        srcb, dstb, rows, gi, gg = bufs[b]
        pltpu.make_async_copy(src_hbm.at[pl.ds(0, K)], srcb, gi).wait()
        pltpu.make_async_copy(dst_hbm.at[pl.ds(0, K)], dstb, gi).wait()
        pltpu.async_copy(y_hbm.at[srcb], rows, gg)

    def gather_wait(b):
        srcb, dstb, rows, gi, gg = bufs[b]
        pltpu.make_async_copy(y_hbm.at[srcb], rows, gg).wait()

    def scatter(b):
        srcb, dstb, rows, gi, gg = bufs[b]
        pltpu.sync_copy(rows, acc.at[dstb], add=True)

    for b in range(NB):
        idx_start(b, b)
    idx_wait_gather_start(0)

    def round3(jj, carry):
        j0 = jj * NB
        for b in range(NB):
            j = j0 + b
            gather_wait(b)

            @pl.when(j + 1 < NCH)
            def _g():
                idx_wait_gather_start((b + 1) % NB)

            scatter(b)

            @pl.when(j + NB < NCH)
            def _i():
                idx_start(j + NB, b)

        return carry

    lax.fori_loop(0, NCH // NB, round3, 0)
    stt = pl.multiple_of(base + NCH * K, 8)
    pltpu.sync_copy(src_hbm.at[pl.ds(stt, TAIL)], srct)
    pltpu.sync_copy(dst_hbm.at[pl.ds(stt, TAIL)], dstt)
    pltpu.async_copy(y_hbm.at[srct], rows0.at[pl.ds(0, TAIL)], gg0).wait()
    pltpu.sync_copy(rows0.at[pl.ds(0, TAIL)], acc.at[dstt], add=True)
    plsc.subcore_barrier()
    pltpu.sync_copy(acc.at[pl.ds(r0, RPT)], out_hbm.at[c, pl.ds(r0, RPT)])

    @pl.when(s == NS - 1)
    def _out_rem():
        pltpu.sync_copy(acc.at[pl.ds(REM_OFF, REM)],
                        out_hbm.at[c, pl.ds(REM_OFF, REM)])


BN = 1000
GRID = N_NODES // BN


def _tc1_body(x_ref, w_ref, dp_ref, y_ref, dinv_ref):
    deg = dp_ref[0] + dp_ref[1] + 1.0
    di = lax.rsqrt(deg)
    dinv_ref[...] = di
    xw = jnp.dot(x_ref[...], w_ref[...], preferred_element_type=jnp.float32)
    y_ref[...] = xw * di


def _tc1(x, W1, deg_parts):
    return pl.pallas_call(
        _tc1_body,
        grid=(GRID,),
        in_specs=[
            pl.BlockSpec((BN, D_IN), lambda i: (i, 0)),
            pl.BlockSpec((D_IN, HID), lambda i: (0, 0)),
            pl.BlockSpec((NC, BN, 1), lambda i: (0, i, 0)),
        ],
        out_specs=[
            pl.BlockSpec((BN, HID), lambda i: (i, 0)),
            pl.BlockSpec((BN, 1), lambda i: (i, 0)),
        ],
        out_shape=[
            jax.ShapeDtypeStruct((N_NODES, HID), jnp.float32),
            jax.ShapeDtypeStruct((N_NODES, 1), jnp.float32),
        ],
    )(x, W1, deg_parts)


def _tc2_body(s_ref, y1_ref, dinv_ref, b1_ref, w2_ref, y2_ref):
    di = dinv_ref[...]
    h = s_ref[0] + s_ref[1] + y1_ref[...]
    h = jnp.maximum(di * h + b1_ref[...][None, :], 0.0)
    y2_ref[...] = di * jnp.dot(h, w2_ref[...], preferred_element_type=jnp.float32)


def _tc2(s1, y1, dinv, b1, W2):
    return pl.pallas_call(
        _tc2_body,
        grid=(GRID,),
        in_specs=[
            pl.BlockSpec((NC, BN, HID), lambda i: (0, i, 0)),
            pl.BlockSpec((BN, HID), lambda i: (i, 0)),
            pl.BlockSpec((BN, 1), lambda i: (i, 0)),
            pl.BlockSpec((HID,), lambda i: (0,)),
            pl.BlockSpec((HID, HID), lambda i: (0, 0)),
        ],
        out_specs=pl.BlockSpec((BN, HID), lambda i: (i, 0)),
        out_shape=jax.ShapeDtypeStruct((N_NODES, HID), jnp.float32),
    )(s1, y1, dinv, b1, W2)


def _tc3_body(s_ref, y2_ref, dinv_ref, b2_ref, batch_ref, wl_ref, bl_ref,
              out_ref, acc_ref):
    i = pl.program_id(0)

    @pl.when(i == 0)
    def _zero():
        acc_ref[...] = jnp.zeros_like(acc_ref)

    di = dinv_ref[...]
    h2 = di * (s_ref[0] + s_ref[1] + y2_ref[...]) + b2_ref[...][None, :]
    bt = batch_ref[0, 0, :]
    m = (bt[None, :] == lax.broadcasted_iota(jnp.int32, (N_GRP, BN), 0))
    acc_ref[...] += jnp.dot(m.astype(jnp.float32), h2,
                            preferred_element_type=jnp.float32,
                            precision=lax.Precision.HIGHEST)

    @pl.when(i == GRID - 1)
    def _final():
        out_ref[...] = (jnp.dot(acc_ref[...], wl_ref[...],
                                preferred_element_type=jnp.float32)
                        + bl_ref[...][None, :])


def _tc3(s2, y2, dinv, b2, batch2d, Wl, bl):
    return pl.pallas_call(
        _tc3_body,
        grid=(GRID,),
        in_specs=[
            pl.BlockSpec((NC, BN, HID), lambda i: (0, i, 0)),
            pl.BlockSpec((BN, HID), lambda i: (i, 0)),
            pl.BlockSpec((BN, 1), lambda i: (i, 0)),
            pl.BlockSpec((HID,), lambda i: (0,)),
            pl.BlockSpec((1, 1, BN), lambda i: (i, 0, 0)),
            pl.BlockSpec((HID, N_CLS), lambda i: (0, 0)),
            pl.BlockSpec((N_CLS,), lambda i: (0,)),
        ],
        out_specs=pl.BlockSpec((N_GRP, N_CLS), lambda i: (0, 0)),
        out_shape=jax.ShapeDtypeStruct((N_GRP, N_CLS), jnp.float32),
        scratch_shapes=[pltpu.VMEM((N_GRP, HID), jnp.float32)],
    )(s2, y2, dinv, b2, batch2d, Wl, bl)


def kernel(x, edge_index, batch, W1, b1, W2, b2, Wl, bl):
    src = edge_index[0]
    dst = edge_index[1]
    zeros_nh = jnp.zeros((N_NODES, HID), jnp.float32)

    deg_parts = _sc_deg(dst).reshape(NC, N_NODES, 1)
    y1, dinv = _tc1(x, W1, deg_parts)
    s1 = _sc_scatter(src, dst, y1, zeros_nh)
    y2 = _tc2(s1, y1, dinv, b1, W2)
    s2 = _sc_scatter(src, dst, y2, zeros_nh)
    return _tc3(s2, y2, dinv, b2, batch.reshape(GRID, 1, BN), Wl, bl)
